# baseline (speedup 1.0000x reference)
import functools

import jax
import jax.numpy as jnp
from jax import lax
from jax.experimental import pallas as pl
from jax.experimental.pallas import tpu as pltpu
from jax.experimental.pallas import tpu_sc as plsc

_NC = 2
_NS = 16
_NW = _NC * _NS
_SUB = 16384
_LANE = 16


def _sc_add_body(x_hbm, t_hbm, o_hbm, xv, tv, *, batch, per_batch):
    wid = lax.axis_index("s") * _NC + lax.axis_index("c")
    nb = batch // _NW

    def batch_body(ib, _):
        b = wid * nb + ib
        def sub_body(k, _):
            base = b * per_batch + k * _SUB
            pltpu.sync_copy(x_hbm.at[pl.ds(base, _SUB)], xv)
            pltpu.sync_copy(t_hbm.at[pl.ds(k * _SUB, _SUB)], tv)
            def add_body(i, _):
                off = i * _LANE
                xv[pl.ds(off, _LANE)] = xv[pl.ds(off, _LANE)] + tv[pl.ds(off, _LANE)]
                return 0
            lax.fori_loop(0, _SUB // _LANE, add_body, 0)
            pltpu.sync_copy(xv, o_hbm.at[pl.ds(base, _SUB)])
            return 0
        lax.fori_loop(0, per_batch // _SUB, sub_body, 0)
        return 0
    lax.fori_loop(0, nb, batch_body, 0)


def kernel(x, table):
    batch, num_patches, proj_dim = x.shape
    per_batch = num_patches * proj_dim
    sc_add = functools.partial(
        pl.kernel,
        out_type=jax.ShapeDtypeStruct((batch * per_batch,), x.dtype),
        mesh=plsc.VectorSubcoreMesh(core_axis_name="c", subcore_axis_name="s"),
        scratch_types=[
            pltpu.VMEM((_SUB,), jnp.float32),
            pltpu.VMEM((_SUB,), jnp.float32),
        ],
    )(functools.partial(_sc_add_body, batch=batch, per_batch=per_batch))
    out = sc_add(x.reshape(-1), table.reshape(-1))
    return out.reshape(x.shape)


# speedup vs baseline: 9.6524x; 9.6524x over previous
import jax
import jax.numpy as jnp
from jax.experimental import pallas as pl


def _add_kernel(x_ref, t_ref, o_ref):
    o_ref[...] = x_ref[...] + t_ref[...][None, :, :]


def kernel(x, table):
    batch, num_patches, proj_dim = x.shape
    block_b = 4
    grid = (batch // block_b,)
    return pl.pallas_call(
        _add_kernel,
        grid=grid,
        in_specs=[
            pl.BlockSpec((block_b, num_patches, proj_dim), lambda b: (b, 0, 0)),
            pl.BlockSpec((num_patches, proj_dim), lambda b: (0, 0)),
        ],
        out_specs=pl.BlockSpec((block_b, num_patches, proj_dim), lambda b: (b, 0, 0)),
        out_shape=jax.ShapeDtypeStruct(x.shape, x.dtype),
    )(x, table)
